# Initial kernel scaffold; baseline (speedup 1.0000x reference)
import jax
import jax.numpy as jnp
from jax.experimental import pallas as pl


def kernel(released, i):
    raise NotImplementedError("write your pallas kernel here")


# speedup vs baseline: 2.2964x; 2.2964x over previous
import jax
import jax.numpy as jnp
from jax import lax
from jax.experimental import pallas as pl
from jax.experimental.pallas import tpu as pltpu
from jax.experimental.pallas import tpu_sc as plsc

_NCELLS = 14
_TPTS = 100000
_BATCH = 16384
_LANES = 16
_IDX_CHUNK = 8192
_UNROLL = 8


def _gather_body(released_hbm, i_hbm, out_hbm, row_v, idx_v, out_v):
    c = lax.axis_index("c")
    s = lax.axis_index("s")
    wid = s * 2 + c

    @pl.when(wid < _NCELLS)
    def _():
        pltpu.sync_copy(released_hbm.at[wid], row_v)
        for h in range(_BATCH // _IDX_CHUNK):
            pltpu.sync_copy(i_hbm.at[pl.ds(h * _IDX_CHUNK, _IDX_CHUNK)], idx_v)

            def step(b, carry):
                base = b * (_UNROLL * _LANES)
                for u in range(_UNROLL):
                    off = base + u * _LANES
                    idx = idx_v[pl.ds(off, _LANES)]
                    vals = plsc.load_gather(row_v, [idx])
                    out_v[pl.ds(h * _IDX_CHUNK + off, _LANES)] = vals
                return carry

            lax.fori_loop(0, _IDX_CHUNK // (_UNROLL * _LANES), step, 0)
        pltpu.sync_copy(out_v, out_hbm.at[wid])


def kernel(released, i):
    mesh = plsc.VectorSubcoreMesh(
        core_axis_name="c", subcore_axis_name="s", num_cores=2, num_subcores=16
    )
    f = pl.kernel(
        _gather_body,
        out_type=jax.ShapeDtypeStruct((_NCELLS, _BATCH), jnp.float32),
        mesh=mesh,
        compiler_params=pltpu.CompilerParams(needs_layout_passes=False),
        scratch_types=[
            pltpu.VMEM((_TPTS,), jnp.float32),
            pltpu.VMEM((_IDX_CHUNK,), jnp.int32),
            pltpu.VMEM((_BATCH,), jnp.float32),
        ],
    )
    return f(released, i.astype(jnp.int32))


# speedup vs baseline: 2.3058x; 1.0041x over previous
import jax
import jax.numpy as jnp
from jax import lax
from jax.experimental import pallas as pl
from jax.experimental.pallas import tpu as pltpu
from jax.experimental.pallas import tpu_sc as plsc

_NCELLS = 14
_TPTS = 100000
_BATCH = 16384
_LANES = 16
_IDX_CHUNK = 8192
_UNROLL = 8


def _gather_body(released_hbm, i_hbm, out_hbm, row_v, idx_v, out_v):
    c = lax.axis_index("c")
    s = lax.axis_index("s")
    wid = s * 2 + c

    @pl.when(wid < _NCELLS)
    def _():
        with jax.named_scope("row_dma"):
            pltpu.sync_copy(released_hbm.at[wid], row_v)
        for h in range(_BATCH // _IDX_CHUNK):
            with jax.named_scope("idx_dma"):
                pltpu.sync_copy(i_hbm.at[pl.ds(h * _IDX_CHUNK, _IDX_CHUNK)], idx_v)

            with jax.named_scope("gather"):
                def step(b, carry):
                    base = b * (_UNROLL * _LANES)
                    for u in range(_UNROLL):
                        off = base + u * _LANES
                        idx = idx_v[pl.ds(off, _LANES)]
                        vals = plsc.load_gather(row_v, [idx])
                        out_v[pl.ds(h * _IDX_CHUNK + off, _LANES)] = vals
                    return carry

                lax.fori_loop(0, _IDX_CHUNK // (_UNROLL * _LANES), step, 0)
        with jax.named_scope("out_dma"):
            pltpu.sync_copy(out_v, out_hbm.at[wid])


def kernel(released, i):
    mesh = plsc.VectorSubcoreMesh(
        core_axis_name="c", subcore_axis_name="s", num_cores=2, num_subcores=16
    )
    f = pl.kernel(
        _gather_body,
        out_type=jax.ShapeDtypeStruct((_NCELLS, _BATCH), jnp.float32),
        mesh=mesh,
        compiler_params=pltpu.CompilerParams(needs_layout_passes=False),
        scratch_types=[
            pltpu.VMEM((_TPTS,), jnp.float32),
            pltpu.VMEM((_IDX_CHUNK,), jnp.int32),
            pltpu.VMEM((_BATCH,), jnp.float32),
        ],
    )
    return f(released, i.astype(jnp.int32))


# speedup vs baseline: 2.6772x; 1.1611x over previous
import jax
import jax.numpy as jnp
from jax import lax
from jax.experimental import pallas as pl
from jax.experimental.pallas import tpu as pltpu
from jax.experimental.pallas import tpu_sc as plsc

_NCELLS = 14
_TPTS = 100000
_BATCH = 16384
_LANES = 16
_HALF = _BATCH // 2
_ROW_STREAMS = 4


def _gather_body(released_hbm, i_hbm, out_hbm, row_v, idx_v, out_v, sem):
    c = lax.axis_index("c")
    s = lax.axis_index("s")
    wid = s * 2 + c

    @pl.when(wid < _NCELLS)
    def _():
        row_cp = pltpu.async_copy(released_hbm.at[wid], row_v, sem)
        idx_cp = pltpu.async_copy(i_hbm, idx_v, sem)
        row_cp.wait()
        idx_cp.wait()

        for h in range(2):
            @plsc.parallel_loop(0, _HALF, step=_LANES, unroll=8)
            def _gather(off):
                idx = idx_v[pl.ds(h * _HALF + off, _LANES)]
                out_v[pl.ds(off, _LANES)] = plsc.load_gather(row_v, [idx])

            pltpu.sync_copy(out_v, out_hbm.at[wid].at[pl.ds(h * _HALF, _HALF)])


def kernel(released, i):
    mesh = plsc.VectorSubcoreMesh(
        core_axis_name="c", subcore_axis_name="s", num_cores=2, num_subcores=16
    )
    f = pl.kernel(
        _gather_body,
        out_type=jax.ShapeDtypeStruct((_NCELLS, _BATCH), jnp.float32),
        mesh=mesh,
        compiler_params=pltpu.CompilerParams(needs_layout_passes=False),
        scratch_types=[
            pltpu.VMEM((_TPTS,), jnp.float32),
            pltpu.VMEM((_BATCH,), jnp.int32),
            pltpu.VMEM((_HALF,), jnp.float32),
            pltpu.SemaphoreType.DMA,
        ],
    )
    return f(released, i.astype(jnp.int32))


# speedup vs baseline: 2.7362x; 1.0220x over previous
import jax
import jax.numpy as jnp
from jax import lax
from jax.experimental import pallas as pl
from jax.experimental.pallas import tpu as pltpu
from jax.experimental.pallas import tpu_sc as plsc

_NCELLS = 14
_TPTS = 100000
_BATCH = 16384
_LANES = 16
_HALF = _BATCH // 2
_UNROLL = 4


def _gather_body(released_hbm, if_hbm, out_hbm, row_v, buf_v, sem):
    c = lax.axis_index("c")
    s = lax.axis_index("s")
    wid = s * 2 + c

    @pl.when(wid < _NCELLS)
    def _():
        row_cp = pltpu.async_copy(released_hbm.at[wid], row_v, sem)
        idx_cp = pltpu.async_copy(if_hbm, buf_v, sem)
        row_cp.wait()
        idx_cp.wait()

        out_row = out_hbm.at[wid]
        half_cps = []
        for h in range(2):
            @plsc.parallel_loop(0, _HALF, step=_LANES, unroll=_UNROLL)
            def _gather(off):
                pos = h * _HALF + off
                idx = plsc.bitcast(buf_v[pl.ds(pos, _LANES)], jnp.int32)
                buf_v[pl.ds(pos, _LANES)] = plsc.load_gather(row_v, [idx])

            half_cps.append(
                pltpu.async_copy(
                    buf_v.at[pl.ds(h * _HALF, _HALF)],
                    out_row.at[pl.ds(h * _HALF, _HALF)],
                    sem,
                )
            )
        for cp in half_cps:
            cp.wait()


def kernel(released, i):
    mesh = plsc.VectorSubcoreMesh(
        core_axis_name="c", subcore_axis_name="s", num_cores=2, num_subcores=16
    )
    f = pl.kernel(
        _gather_body,
        out_type=jax.ShapeDtypeStruct((_NCELLS, _BATCH), jnp.float32),
        mesh=mesh,
        compiler_params=pltpu.CompilerParams(needs_layout_passes=False),
        scratch_types=[
            pltpu.VMEM((_TPTS,), jnp.float32),
            pltpu.VMEM((_BATCH,), jnp.float32),
            pltpu.SemaphoreType.DMA,
        ],
    )
    i_f = lax.bitcast_convert_type(i.astype(jnp.int32), jnp.float32)
    return f(released, i_f)


# speedup vs baseline: 2.8255x; 1.0326x over previous
import jax
import jax.numpy as jnp
from jax import lax
from jax.experimental import pallas as pl
from jax.experimental.pallas import tpu as pltpu
from jax.experimental.pallas import tpu_sc as plsc

_NCELLS = 14
_TPTS = 100000
_BATCH = 16384
_LANES = 16
_BH = _BATCH // 2
_QTR = _BH // 2
_UNROLL = 4


def _gather_body(released_hbm, if_hbm, out_hbm, row_v, buf_v, sem):
    c = lax.axis_index("c")
    s = lax.axis_index("s")
    wid = s * 2 + c

    @pl.when(wid < 2 * _NCELLS)
    def _():
        cell = wid // 2
        bh = wid % 2
        row_cp = pltpu.async_copy(released_hbm.at[cell], row_v, sem)
        idx_cp = pltpu.async_copy(if_hbm.at[pl.ds(bh * _BH, _BH)], buf_v, sem)
        row_cp.wait()
        idx_cp.wait()

        out_row = out_hbm.at[cell]
        half_cps = []
        for h in range(2):
            @plsc.parallel_loop(0, _QTR, step=_LANES, unroll=_UNROLL)
            def _gather(off):
                pos = h * _QTR + off
                idx = plsc.bitcast(buf_v[pl.ds(pos, _LANES)], jnp.int32)
                buf_v[pl.ds(pos, _LANES)] = plsc.load_gather(row_v, [idx])

            half_cps.append(
                pltpu.async_copy(
                    buf_v.at[pl.ds(h * _QTR, _QTR)],
                    out_row.at[pl.ds(bh * _BH + h * _QTR, _QTR)],
                    sem,
                )
            )
        for cp in half_cps:
            cp.wait()


def kernel(released, i):
    mesh = plsc.VectorSubcoreMesh(
        core_axis_name="c", subcore_axis_name="s", num_cores=2, num_subcores=16
    )
    f = pl.kernel(
        _gather_body,
        out_type=jax.ShapeDtypeStruct((_NCELLS, _BATCH), jnp.float32),
        mesh=mesh,
        compiler_params=pltpu.CompilerParams(needs_layout_passes=False),
        scratch_types=[
            pltpu.VMEM((_TPTS,), jnp.float32),
            pltpu.VMEM((_BH,), jnp.float32),
            pltpu.SemaphoreType.DMA,
        ],
    )
    i_f = lax.bitcast_convert_type(i.astype(jnp.int32), jnp.float32)
    return f(released, i_f)
